# Initial kernel scaffold; baseline (speedup 1.0000x reference)
import jax
import jax.numpy as jnp
from jax.experimental import pallas as pl


def kernel(x, edge_index, Wc, bc, Wr, br, gamma, beta, rmean, rvar, Win, b_in, Wout, b_out):
    raise NotImplementedError("write your pallas kernel here")


# speedup vs baseline: 2.8507x; 2.8507x over previous
import functools

import jax
import jax.numpy as jnp
from jax import lax
from jax.experimental import pallas as pl
from jax.experimental.pallas import tpu as pltpu
from jax.experimental.pallas import tpu_sc as plsc

N_NODES = 10000
N_EDGES = 320000
D = 128
N_LAYERS = 12

NC = 2
NS = 16
NW = NC * NS
NPAD = 10240
ROWS_PER_S = NPAD // NS
CH = 128
NCHUNK = 80
E_PER_W = NCHUNK * CH
E_PAD = NW * E_PER_W
DEGW = 16

_mesh = plsc.VectorSubcoreMesh(core_axis_name="c", subcore_axis_name="s")


def _make_sc_msg(dd):
  @functools.partial(
      pl.kernel,
      out_type=jax.ShapeDtypeStruct((NC, NPAD, dd), jnp.float32),
      mesh=_mesh,
      scratch_types=[
          pltpu.VMEM((NCHUNK, CH), jnp.int32),
          pltpu.VMEM((2, CH), jnp.int32),
          pltpu.VMEM((2, CH, dd), jnp.float32),
          pltpu.VMEM_SHARED((NPAD, dd), jnp.float32),
          pltpu.SemaphoreType.DMA,
          pltpu.SemaphoreType.DMA,
          pltpu.SemaphoreType.DMA,
          pltpu.SemaphoreType.DMA,
      ],
  )
  def _sc_msg(h_hbm, srcw_hbm, dstw_hbm, zeros_hbm, out_hbm,
              src_v, dst_v, rows_v, acc_sh, g0, g1, d0, d1):
    c = lax.axis_index("c")
    s = lax.axis_index("s")
    wid = s * NC + c
    pltpu.sync_copy(zeros_hbm.at[pl.ds(s * ROWS_PER_S, ROWS_PER_S)],
                    acc_sh.at[pl.ds(s * ROWS_PER_S, ROWS_PER_S)])
    pltpu.sync_copy(srcw_hbm.at[wid], src_v)
    pltpu.sync_copy(dstw_hbm.at[wid, 0], dst_v.at[0])
    plsc.subcore_barrier()
    pltpu.async_copy(h_hbm.at[src_v.at[0]], rows_v.at[0], g0)

    def pair(t, carry):
        j0 = 2 * t
        j1 = j0 + 1
        pltpu.async_copy(dstw_hbm.at[wid, j1], dst_v.at[1], d1)
        pltpu.async_copy(h_hbm.at[src_v.at[j1]], rows_v.at[1], g1)
        pltpu.make_async_copy(h_hbm.at[src_v.at[j0]], rows_v.at[0], g0).wait()

        @pl.when(t > 0)
        def _():
            pltpu.make_async_copy(dstw_hbm.at[wid, j0], dst_v.at[0], d0).wait()

        pltpu.sync_copy(rows_v.at[0], acc_sh.at[dst_v.at[0]], add=True)

        @pl.when(j0 + 2 < NCHUNK)
        def _():
            pltpu.async_copy(dstw_hbm.at[wid, j0 + 2], dst_v.at[0], d0)
            pltpu.async_copy(h_hbm.at[src_v.at[j0 + 2]], rows_v.at[0], g0)

        pltpu.make_async_copy(h_hbm.at[src_v.at[j1]], rows_v.at[1], g1).wait()
        pltpu.make_async_copy(dstw_hbm.at[wid, j1], dst_v.at[1], d1).wait()
        pltpu.sync_copy(rows_v.at[1], acc_sh.at[dst_v.at[1]], add=True)
        return carry

    lax.fori_loop(0, NCHUNK // 2, pair, 0)
    plsc.subcore_barrier()
    pltpu.sync_copy(acc_sh.at[pl.ds(s * ROWS_PER_S, ROWS_PER_S)],
                    out_hbm.at[c, pl.ds(s * ROWS_PER_S, ROWS_PER_S)])

  return _sc_msg


_sc_msg = _make_sc_msg(D)


@functools.partial(
    pl.kernel,
    out_type=jax.ShapeDtypeStruct((NC, NPAD, D), jnp.float32),
    mesh=_mesh,
    scratch_types=[
        pltpu.VMEM((NCHUNK, CH), jnp.int32),
        pltpu.VMEM((CH, D), jnp.float32),
        pltpu.VMEM_SHARED((NPAD, D), jnp.float32),
        pltpu.SemaphoreType.DMA,
        pltpu.SemaphoreType.DMA,
    ],
)
def _sc_deg(dstw_hbm, ones_hbm, zeros_hbm, out_hbm, dst_v, ones_v, acc_sh,
            sa, sb):
    c = lax.axis_index("c")
    s = lax.axis_index("s")
    wid = s * NC + c
    pltpu.sync_copy(zeros_hbm.at[pl.ds(s * ROWS_PER_S, ROWS_PER_S)],
                    acc_sh.at[pl.ds(s * ROWS_PER_S, ROWS_PER_S)])
    pltpu.sync_copy(dstw_hbm.at[wid], dst_v)
    pltpu.sync_copy(ones_hbm, ones_v)
    plsc.subcore_barrier()

    def pair(t, carry):
        j0 = 2 * t
        j1 = j0 + 1

        @pl.when(t > 0)
        def _():
            pltpu.make_async_copy(ones_v, acc_sh.at[dst_v.at[j0 - 2]],
                                  sa).wait()

        pltpu.async_copy(ones_v, acc_sh.at[dst_v.at[j0]], sa, add=True)

        @pl.when(t > 0)
        def _():
            pltpu.make_async_copy(ones_v, acc_sh.at[dst_v.at[j1 - 2]],
                                  sb).wait()

        pltpu.async_copy(ones_v, acc_sh.at[dst_v.at[j1]], sb, add=True)
        return carry

    lax.fori_loop(0, NCHUNK // 2, pair, 0)
    pltpu.make_async_copy(ones_v, acc_sh.at[dst_v.at[NCHUNK - 2]], sa).wait()
    pltpu.make_async_copy(ones_v, acc_sh.at[dst_v.at[NCHUNK - 1]], sb).wait()
    plsc.subcore_barrier()
    pltpu.sync_copy(acc_sh.at[pl.ds(s * ROWS_PER_S, ROWS_PER_S)],
                    out_hbm.at[c, pl.ds(s * ROWS_PER_S, ROWS_PER_S)])


_R = 1024
_G = NPAD // _R


def _invd_body(degp_r, o_r):
    s = degp_r[0] + degp_r[1]
    o_r[...] = 1.0 / jnp.maximum(s[:, :1], 1.0)


def _tc_invdeg(degp):
    return pl.pallas_call(
        _invd_body,
        grid=(_G,),
        in_specs=[pl.BlockSpec((NC, _R, D), lambda i: (0, i, 0))],
        out_specs=pl.BlockSpec((_R, 1), lambda i: (i, 0)),
        out_shape=jax.ShapeDtypeStruct((NPAD, 1), jnp.float32),
    )(degp)


def _dense_body(m_r, h_r, invd_r, wc_r, bc_r, wr_r, br_r, a_r, b_r, o_r):
    m = (m_r[0] + m_r[1]) * invd_r[...]
    conv = lax.dot_general(m, wc_r[...], (((1,), (0,)), ((), ())),
                           preferred_element_type=jnp.float32,
                           precision=lax.Precision.HIGHEST)
    conv = jnp.maximum(conv + bc_r[...], 0.0)
    res = lax.dot_general(h_r[...], wr_r[...], (((1,), (0,)), ((), ())),
                          preferred_element_type=jnp.float32,
                          precision=lax.Precision.HIGHEST)
    res = jnp.maximum(res + br_r[...], 0.0)
    o_r[...] = (conv + res) * a_r[...] + b_r[...]


def _tc_dense(m_partials, h, invd, wc, bc, wr, br, a, b):
    row = lambda i: (i, 0)
    fixw = lambda i: (0, 0)
    return pl.pallas_call(
        _dense_body,
        grid=(_G,),
        in_specs=[
            pl.BlockSpec((NC, _R, D), lambda i: (0, i, 0)),
            pl.BlockSpec((_R, D), row),
            pl.BlockSpec((_R, 1), row),
            pl.BlockSpec((D, D), fixw),
            pl.BlockSpec((1, D), fixw),
            pl.BlockSpec((D, D), fixw),
            pl.BlockSpec((1, D), fixw),
            pl.BlockSpec((1, D), fixw),
            pl.BlockSpec((1, D), fixw),
        ],
        out_specs=pl.BlockSpec((_R, D), row),
        out_shape=jax.ShapeDtypeStruct((NPAD, D), jnp.float32),
    )(m_partials, h, invd, wc, bc, wr, br, a, b)


def _readout_body(h_r, win_r, bin_r, wout_r, bout_r, o_r):
    i = pl.program_id(0)
    z = lax.dot_general(h_r[...], win_r[...], (((1,), (0,)), ((), ())),
                        preferred_element_type=jnp.float32,
                        precision=lax.Precision.HIGHEST)
    z = jnp.maximum(z + bin_r[...], 0.0)
    rows = lax.broadcasted_iota(jnp.int32, (_R, D), 0) + i * _R
    z = jnp.where(rows < N_NODES, z, 0.0)
    part = jnp.sum(z, axis=0, keepdims=True)

    @pl.when(i == 0)
    def _():
        o_r[...] = jnp.zeros_like(o_r)

    o_r[...] += part

    @pl.when(i == _G - 1)
    def _():
        o_r[...] = lax.dot_general(
            o_r[...] * (1.0 / N_NODES), wout_r[...], (((1,), (0,)), ((), ())),
            preferred_element_type=jnp.float32,
            precision=lax.Precision.HIGHEST) + bout_r[...]


def _tc_readout(h, win, b_in, wout, b_out):
    fixw = lambda i: (0, 0)
    return pl.pallas_call(
        _readout_body,
        grid=(_G,),
        in_specs=[
            pl.BlockSpec((_R, D), lambda i: (i, 0)),
            pl.BlockSpec((D, D), fixw),
            pl.BlockSpec((1, D), fixw),
            pl.BlockSpec((D, D), fixw),
            pl.BlockSpec((1, D), fixw),
        ],
        out_specs=pl.BlockSpec((1, D), fixw),
        out_shape=jax.ShapeDtypeStruct((1, D), jnp.float32),
    )(h, win, b_in, wout, b_out)


def kernel(x, edge_index, Wc, bc, Wr, br, gamma, beta, rmean, rvar,
           Win, b_in, Wout, b_out):
    npad_e = E_PAD - N_EDGES
    src = jnp.concatenate(
        [edge_index[0].astype(jnp.int32),
         jnp.zeros((npad_e,), jnp.int32)]).reshape(NW, NCHUNK, CH)
    dst = jnp.concatenate(
        [edge_index[1].astype(jnp.int32),
         jnp.full((npad_e,), N_NODES, jnp.int32)]).reshape(NW, NCHUNK, CH)

    zeros = jnp.zeros((NPAD, D), jnp.float32)
    ones_ch = jnp.ones((CH, D), jnp.float32)

    degp = _sc_deg(dst, ones_ch, zeros)
    invd = _tc_invdeg(degp)

    a_all = gamma / jnp.sqrt(rvar + 1e-5)
    b_all = beta - rmean * a_all

    h = jnp.concatenate([x, jnp.zeros((NPAD - N_NODES, D), jnp.float32)],
                        axis=0)
    for l in range(N_LAYERS):
        mp = _sc_msg(h, src, dst, zeros)
        h = _tc_dense(mp, h, invd,
                      Wc[l], bc[l].reshape(1, D),
                      Wr[l], br[l].reshape(1, D),
                      a_all[l].reshape(1, D), b_all[l].reshape(1, D))

    return _tc_readout(h, Win, b_in.reshape(1, D), Wout, b_out.reshape(1, D))


# speedup vs baseline: 2.9765x; 1.0441x over previous
import functools

import jax
import jax.numpy as jnp
from jax import lax
from jax.experimental import pallas as pl
from jax.experimental.pallas import tpu as pltpu
from jax.experimental.pallas import tpu_sc as plsc

N_NODES = 10000
N_EDGES = 320000
D = 128
N_LAYERS = 12

NC = 2
NS = 16
NW = NC * NS
NPAD = 10240
ROWS_PER_S = NPAD // NS
CH = 128
NCHUNK = 80
E_PER_W = NCHUNK * CH
E_PAD = NW * E_PER_W

_mesh = plsc.VectorSubcoreMesh(core_axis_name="c", subcore_axis_name="s")


def _make_sc_msg(dd):
  @functools.partial(
      pl.kernel,
      out_type=jax.ShapeDtypeStruct((NC, NPAD, dd), jnp.float32),
      mesh=_mesh,
      scratch_types=[
          pltpu.VMEM((NCHUNK, CH), jnp.int32),
          pltpu.VMEM((2, CH), jnp.int32),
          pltpu.VMEM((2, CH, dd), jnp.float32),
          pltpu.VMEM_SHARED((NPAD, dd), jnp.float32),
          pltpu.SemaphoreType.DMA,
          pltpu.SemaphoreType.DMA,
          pltpu.SemaphoreType.DMA,
          pltpu.SemaphoreType.DMA,
      ],
  )
  def _sc_msg(h_hbm, srcw_hbm, dstw_hbm, zeros_hbm, out_hbm,
              src_v, dst_v, rows_v, acc_sh, g0, g1, d0, d1):
    c = lax.axis_index("c")
    s = lax.axis_index("s")
    wid = s * NC + c
    pltpu.sync_copy(zeros_hbm.at[pl.ds(s * ROWS_PER_S, ROWS_PER_S)],
                    acc_sh.at[pl.ds(s * ROWS_PER_S, ROWS_PER_S)])
    pltpu.sync_copy(srcw_hbm.at[wid], src_v)
    pltpu.sync_copy(dstw_hbm.at[wid, 0], dst_v.at[0])
    plsc.subcore_barrier()
    pltpu.async_copy(h_hbm.at[src_v.at[0]], rows_v.at[0], g0)

    def pair(t, carry):
        j0 = 2 * t
        j1 = j0 + 1
        pltpu.async_copy(dstw_hbm.at[wid, j1], dst_v.at[1], d1)
        pltpu.async_copy(h_hbm.at[src_v.at[j1]], rows_v.at[1], g1)
        pltpu.make_async_copy(h_hbm.at[src_v.at[j0]], rows_v.at[0], g0).wait()

        @pl.when(t > 0)
        def _():
            pltpu.make_async_copy(dstw_hbm.at[wid, j0], dst_v.at[0], d0).wait()

        pltpu.sync_copy(rows_v.at[0], acc_sh.at[dst_v.at[0]], add=True)

        @pl.when(j0 + 2 < NCHUNK)
        def _():
            pltpu.async_copy(dstw_hbm.at[wid, j0 + 2], dst_v.at[0], d0)
            pltpu.async_copy(h_hbm.at[src_v.at[j0 + 2]], rows_v.at[0], g0)

        pltpu.make_async_copy(h_hbm.at[src_v.at[j1]], rows_v.at[1], g1).wait()
        pltpu.make_async_copy(dstw_hbm.at[wid, j1], dst_v.at[1], d1).wait()
        pltpu.sync_copy(rows_v.at[1], acc_sh.at[dst_v.at[1]], add=True)
        return carry

    lax.fori_loop(0, NCHUNK // 2, pair, 0)
    plsc.subcore_barrier()
    pltpu.sync_copy(acc_sh.at[pl.ds(s * ROWS_PER_S, ROWS_PER_S)],
                    out_hbm.at[c, pl.ds(s * ROWS_PER_S, ROWS_PER_S)])

  return _sc_msg


_sc_msg = _make_sc_msg(D)


_R = 1024
_G = NPAD // _R


def _invd_body(degp_r, o_r):
    s = degp_r[0] + degp_r[1]
    o_r[...] = 1.0 / jnp.maximum(s[:, :1], 1.0)


def _tc_invdeg(degp):
    return pl.pallas_call(
        _invd_body,
        grid=(_G,),
        in_specs=[pl.BlockSpec((NC, _R, D), lambda i: (0, i, 0))],
        out_specs=pl.BlockSpec((_R, 1), lambda i: (i, 0)),
        out_shape=jax.ShapeDtypeStruct((NPAD, 1), jnp.float32),
    )(degp)


def _dense_body(m_r, h_r, invd_r, wc_r, bc_r, wr_r, br_r, a_r, b_r, o_r):
    m = (m_r[0] + m_r[1]) * invd_r[...]
    conv = lax.dot_general(m, wc_r[...], (((1,), (0,)), ((), ())),
                           preferred_element_type=jnp.float32,
                           precision=lax.Precision.HIGHEST)
    conv = jnp.maximum(conv + bc_r[...], 0.0)
    res = lax.dot_general(h_r[...], wr_r[...], (((1,), (0,)), ((), ())),
                          preferred_element_type=jnp.float32,
                          precision=lax.Precision.HIGHEST)
    res = jnp.maximum(res + br_r[...], 0.0)
    o_r[...] = (conv + res) * a_r[...] + b_r[...]


def _tc_dense(m_partials, h, invd, wc, bc, wr, br, a, b):
    row = lambda i: (i, 0)
    fixw = lambda i: (0, 0)
    return pl.pallas_call(
        _dense_body,
        grid=(_G,),
        in_specs=[
            pl.BlockSpec((NC, _R, D), lambda i: (0, i, 0)),
            pl.BlockSpec((_R, D), row),
            pl.BlockSpec((_R, 1), row),
            pl.BlockSpec((D, D), fixw),
            pl.BlockSpec((1, D), fixw),
            pl.BlockSpec((D, D), fixw),
            pl.BlockSpec((1, D), fixw),
            pl.BlockSpec((1, D), fixw),
            pl.BlockSpec((1, D), fixw),
        ],
        out_specs=pl.BlockSpec((_R, D), row),
        out_shape=jax.ShapeDtypeStruct((NPAD, D), jnp.float32),
    )(m_partials, h, invd, wc, bc, wr, br, a, b)


def _readout_body(h_r, win_r, bin_r, wout_r, bout_r, o_r):
    i = pl.program_id(0)
    z = lax.dot_general(h_r[...], win_r[...], (((1,), (0,)), ((), ())),
                        preferred_element_type=jnp.float32,
                        precision=lax.Precision.HIGHEST)
    z = jnp.maximum(z + bin_r[...], 0.0)
    rows = lax.broadcasted_iota(jnp.int32, (_R, D), 0) + i * _R
    z = jnp.where(rows < N_NODES, z, 0.0)
    part = jnp.sum(z, axis=0, keepdims=True)

    @pl.when(i == 0)
    def _():
        o_r[...] = jnp.zeros_like(o_r)

    o_r[...] += part

    @pl.when(i == _G - 1)
    def _():
        o_r[...] = lax.dot_general(
            o_r[...] * (1.0 / N_NODES), wout_r[...], (((1,), (0,)), ((), ())),
            preferred_element_type=jnp.float32,
            precision=lax.Precision.HIGHEST) + bout_r[...]


def _tc_readout(h, win, b_in, wout, b_out):
    fixw = lambda i: (0, 0)
    return pl.pallas_call(
        _readout_body,
        grid=(_G,),
        in_specs=[
            pl.BlockSpec((_R, D), lambda i: (i, 0)),
            pl.BlockSpec((D, D), fixw),
            pl.BlockSpec((1, D), fixw),
            pl.BlockSpec((D, D), fixw),
            pl.BlockSpec((1, D), fixw),
        ],
        out_specs=pl.BlockSpec((1, D), fixw),
        out_shape=jax.ShapeDtypeStruct((1, D), jnp.float32),
    )(h, win, b_in, wout, b_out)


def kernel(x, edge_index, Wc, bc, Wr, br, gamma, beta, rmean, rvar,
           Win, b_in, Wout, b_out):
    npad_e = E_PAD - N_EDGES
    src = jnp.concatenate(
        [edge_index[0].astype(jnp.int32),
         jnp.zeros((npad_e,), jnp.int32)]).reshape(NW, NCHUNK, CH)
    dst = jnp.concatenate(
        [edge_index[1].astype(jnp.int32),
         jnp.full((npad_e,), N_NODES, jnp.int32)]).reshape(NW, NCHUNK, CH)

    zeros = jnp.zeros((NPAD, D), jnp.float32)
    ones_full = jnp.ones((NPAD, D), jnp.float32)

    degp = _sc_msg(ones_full, src, dst, zeros)
    invd = _tc_invdeg(degp)

    a_all = gamma / jnp.sqrt(rvar + 1e-5)
    b_all = beta - rmean * a_all

    h = jnp.concatenate([x, jnp.zeros((NPAD - N_NODES, D), jnp.float32)],
                        axis=0)
    for l in range(N_LAYERS):
        mp = _sc_msg(h, src, dst, zeros)
        h = _tc_dense(mp, h, invd,
                      Wc[l], bc[l].reshape(1, D),
                      Wr[l], br[l].reshape(1, D),
                      a_all[l].reshape(1, D), b_all[l].reshape(1, D))

    return _tc_readout(h, Win, b_in.reshape(1, D), Wout, b_out.reshape(1, D))
